# baseline (speedup 1.0000x reference)
import functools

import jax
import jax.numpy as jnp
from jax import lax
from jax.experimental import pallas as pl
from jax.experimental.pallas import tpu as pltpu
from jax.experimental.pallas import tpu_sc as plsc

N = 10000
E = 320000
D = 128
FD = 64
NUM_ITER = 2

NC = 2
NS = 16
L = 16
NW = NC * NS
NCHK = 5
EC = E // NCHK
EPW = EC // NW
CH = 80
SUB = 5
CB = CH * SUB
NSTEP = EPW // CB
NP2 = 10240
RPT = NP2 // NS
CH_S = 40
SUB_S = 5
CB_S = CH_S * SUB_S
NSTEP_S = EPW // CB_S
NP1 = 10240
DPT = NP1 // NS
EPW_D = E // NW
RPD = EPW_D // CH


def _sc_deg_body(dst2_ref, z1_ref, deg_out, idx_v, ones_v, deg_sp, sem):
  c = lax.axis_index("c")
  s = lax.axis_index("s")
  wid = s * NC + c
  pltpu.sync_copy(z1_ref.at[pl.ds(s * DPT, DPT)],
                  deg_sp.at[pl.ds(s * DPT, DPT)])
  for j in range(CH // L):
    ones_v[pl.ds(j * L, L)] = jnp.ones((L,), jnp.float32)
  pltpu.sync_copy(dst2_ref.at[wid], idx_v)
  plsc.subcore_barrier()

  def step(i, carry):
    copies = [pltpu.async_copy(ones_v, deg_sp.at[idx_v.at[i, k]],
                               sem, add=True)
              for k in range(SUB)]
    for cp in copies:
      cp.wait()
    return carry

  lax.fori_loop(0, RPD // SUB, step, 0)
  plsc.subcore_barrier()
  pltpu.sync_copy(deg_sp.at[pl.ds(s * DPT, DPT)],
                  deg_out.at[c, pl.ds(s * DPT, DPT)])


@functools.lru_cache(maxsize=None)
def _make_sc_deg():
  mesh = plsc.VectorSubcoreMesh(
      core_axis_name="c", subcore_axis_name="s", num_cores=NC,
      num_subcores=NS)
  return pl.kernel(
      _sc_deg_body,
      out_type=jax.ShapeDtypeStruct((NC, NP1), jnp.float32),
      mesh=mesh,
      scratch_types=[
          pltpu.VMEM((RPD // SUB, SUB, CH), jnp.int32),
          pltpu.VMEM((CH,), jnp.float32),
          pltpu.VMEM_SHARED((NP1,), jnp.float32),
          pltpu.SemaphoreType.DMA,
      ],
      compiler_params=pltpu.CompilerParams(use_tc_tiling_on_sc=True),
  )


def _sc_gather_body(tab_ref, dst_ref, src_ref, u_ref, p_ref,
                    idx_d, idx_s, rows_d, rows_s, sem_g, sem_wd, sem_ws):
  wid = lax.axis_index("s") * NC + lax.axis_index("c")
  base0 = wid * EPW

  def drain(rows, out, sem):
    pltpu.make_async_copy(rows, out.at[pl.ds(0, CB)], sem).wait()

  def step(i, carry):
    base = base0 + i * CB

    @pl.when(i > 0)
    def _():
      drain(rows_d, u_ref, sem_wd)
      drain(rows_s, p_ref, sem_ws)

    pltpu.sync_copy(dst_ref.at[pl.ds(base, CB)], idx_d)
    pltpu.sync_copy(src_ref.at[pl.ds(base, CB)], idx_s)
    gs = [pltpu.async_copy(tab_ref.at[idx_d.at[pl.ds(k * CH, CH)]],
                           rows_d.at[pl.ds(k * CH, CH)], sem_g)
          for k in range(SUB)]
    gs += [pltpu.async_copy(tab_ref.at[idx_s.at[pl.ds(k * CH, CH)]],
                            rows_s.at[pl.ds(k * CH, CH)], sem_g)
           for k in range(SUB)]
    for cp in gs:
      cp.wait()
    pltpu.async_copy(rows_d, u_ref.at[pl.ds(base, CB)], sem_wd)
    pltpu.async_copy(rows_s, p_ref.at[pl.ds(base, CB)], sem_ws)
    return carry

  lax.fori_loop(0, NSTEP, step, 0)
  drain(rows_d, u_ref, sem_wd)
  drain(rows_s, p_ref, sem_ws)


@functools.lru_cache(maxsize=None)
def _make_sc_gather():
  mesh = plsc.VectorSubcoreMesh(
      core_axis_name="c", subcore_axis_name="s", num_cores=NC,
      num_subcores=NS)
  return pl.kernel(
      _sc_gather_body,
      out_type=(jax.ShapeDtypeStruct((EC, D), jnp.float32),
                jax.ShapeDtypeStruct((EC, D), jnp.float32)),
      mesh=mesh,
      scratch_types=[
          pltpu.VMEM((CB,), jnp.int32),
          pltpu.VMEM((CB,), jnp.int32),
          pltpu.VMEM((CB, D), jnp.float32),
          pltpu.VMEM((CB, D), jnp.float32),
          pltpu.SemaphoreType.DMA,
          pltpu.SemaphoreType.DMA,
          pltpu.SemaphoreType.DMA,
      ],
      compiler_params=pltpu.CompilerParams(use_tc_tiling_on_sc=True),
  )


def _sc_scatter_body(msg_ref, dst2_ref, z2_ref, agg_out,
                     idx_v, msg_v, agg_sp, sem):
  c = lax.axis_index("c")
  s = lax.axis_index("s")
  wid = s * NC + c
  pltpu.sync_copy(z2_ref.at[pl.ds(s * RPT, RPT)],
                  agg_sp.at[pl.ds(s * RPT, RPT)])
  plsc.subcore_barrier()

  base0 = wid * EPW

  def step(i, carry):
    base = base0 + i * CB_S
    lds = [pltpu.async_copy(msg_ref.at[pl.ds(base, CB_S)], msg_v, sem),
           pltpu.async_copy(dst2_ref.at[wid, i], idx_v, sem)]
    for cp in lds:
      cp.wait()
    copies = [pltpu.async_copy(msg_v.at[pl.ds(k * CH_S, CH_S)],
                               agg_sp.at[idx_v.at[k]], sem, add=True)
              for k in range(SUB_S)]
    for cp in copies:
      cp.wait()
    return carry

  lax.fori_loop(0, NSTEP_S, step, 0)
  plsc.subcore_barrier()
  pltpu.sync_copy(agg_sp.at[pl.ds(s * RPT, RPT)],
                  agg_out.at[c, pl.ds(s * RPT, RPT)])


@functools.lru_cache(maxsize=None)
def _make_sc_scatter():
  mesh = plsc.VectorSubcoreMesh(
      core_axis_name="c", subcore_axis_name="s", num_cores=NC,
      num_subcores=NS)
  return pl.kernel(
      _sc_scatter_body,
      out_type=jax.ShapeDtypeStruct((NC, NP2, D), jnp.float32),
      mesh=mesh,
      scratch_types=[
          pltpu.VMEM((SUB_S, CH_S), jnp.int32),
          pltpu.VMEM((CB_S, D), jnp.float32),
          pltpu.VMEM_SHARED((NP2, D), jnp.float32),
          pltpu.SemaphoreType.DMA,
      ],
      compiler_params=pltpu.CompilerParams(use_tc_tiling_on_sc=True),
  )


BLK = 2560


def _tc_gate_body(u_ref, p_ref, lnw_ref, lnb_ref, wg_ref, bg_ref, msg_ref):
  u = u_ref[...]
  p = p_ref[...]
  o8 = jnp.ones((D, 8), jnp.float32)
  s1 = jnp.dot(u + p, o8, preferred_element_type=jnp.float32)[:, 0]
  s2 = jnp.dot(u * u + p * p, o8,
               preferred_element_type=jnp.float32)[:, 0]
  mu = s1 / (2.0 * D)
  var = s2 / (2.0 * D) - mu * mu
  inv = lax.rsqrt(var + 1e-5)
  lnw = lnw_ref[...]
  lnb = lnb_ref[...]
  nu = jnp.maximum((u - mu[:, None]) * inv[:, None] * lnw[0] + lnb[0], 0.0)
  np_ = jnp.maximum((p - mu[:, None]) * inv[:, None] * lnw[1] + lnb[1], 0.0)
  wg = wg_ref[...]
  g = (jnp.dot(nu.astype(jnp.bfloat16), wg[:D],
               preferred_element_type=jnp.float32)
       + jnp.dot(np_.astype(jnp.bfloat16), wg[D:],
                 preferred_element_type=jnp.float32)
       + bg_ref[...])
  sg = 1.0 / (1.0 + jnp.exp(-g))
  o8f = jnp.ones((FD, 8), jnp.float32)
  gate = jnp.dot(sg, o8f, preferred_element_type=jnp.float32)[:, 0] / FD
  msg_ref[...] = p * gate[:, None]


def _make_tc_gate():
  return pl.pallas_call(
      _tc_gate_body,
      grid=(EC // BLK,),
      in_specs=[
          pl.BlockSpec((BLK, D), lambda i: (i, 0)),
          pl.BlockSpec((BLK, D), lambda i: (i, 0)),
          pl.BlockSpec((2, D), lambda i: (0, 0)),
          pl.BlockSpec((2, D), lambda i: (0, 0)),
          pl.BlockSpec((2 * D, FD), lambda i: (0, 0)),
          pl.BlockSpec((1, FD), lambda i: (0, 0)),
      ],
      out_specs=pl.BlockSpec((BLK, D), lambda i: (i, 0)),
      out_shape=jax.ShapeDtypeStruct((EC, D), jnp.float32),
  )


RB = 2000


def _tc_update_body(p0, p1, p2, p3, p4, degp_ref, h_ref, wih_ref, whh_ref,
                    b_ref, hn_ref):
  agg = (p0[0] + p0[1] + p1[0] + p1[1] + p2[0] + p2[1]
         + p3[0] + p3[1] + p4[0] + p4[1])
  deg = degp_ref[:, 0] + degp_ref[:, 1]
  aggn = agg / jnp.maximum(deg, 1.0)[:, None]
  h = h_ref[...]
  hn = (jnp.dot(jnp.maximum(aggn, 0.0), wih_ref[...],
                preferred_element_type=jnp.float32)
        + jnp.dot(jnp.maximum(h, 0.0), whh_ref[...],
                  preferred_element_type=jnp.float32)
        + b_ref[0] + b_ref[1])
  hn_ref[...] = hn


def _make_tc_update():
  part_spec = pl.BlockSpec((NC, RB, D), lambda i: (0, i, 0))
  return pl.pallas_call(
      _tc_update_body,
      grid=(N // RB,),
      in_specs=[
          part_spec, part_spec, part_spec, part_spec, part_spec,
          pl.BlockSpec((RB, NC), lambda i: (i, 0)),
          pl.BlockSpec((RB, D), lambda i: (i, 0)),
          pl.BlockSpec((D, D), lambda i: (0, 0)),
          pl.BlockSpec((D, D), lambda i: (0, 0)),
          pl.BlockSpec((2, D), lambda i: (0, 0)),
      ],
      out_specs=pl.BlockSpec((RB, D), lambda i: (i, 0)),
      out_shape=jax.ShapeDtypeStruct((N, D), jnp.float32),
  )


def kernel(x, edge_index, ln_w, ln_b, Wg, bg, Wih, bih, Whh, bhh):
  src = edge_index[0]
  dst = edge_index[1]
  dst_deg = dst.reshape(NW, RPD // SUB, SUB, CH)
  dst_sca = dst.reshape(NCHK, NW, NSTEP_S, SUB_S, CH_S)
  lnw2 = ln_w.reshape(2, D)
  lnb2 = ln_b.reshape(2, D)
  wg_bf = Wg.astype(jnp.bfloat16)
  bg2 = bg.reshape(1, FD)
  b2 = jnp.stack([bih, bhh])
  z2 = jnp.zeros((NP2, D), jnp.float32)
  z1 = jnp.zeros((NP1,), jnp.float32)
  tc_gate = _make_tc_gate()
  tc_update = _make_tc_update()
  sc_gather = _make_sc_gather()
  sc_scatter = _make_sc_scatter()

  degp = _make_sc_deg()(dst_deg, z1)
  degp_t = degp.T[:N]

  h = x
  for _ in range(NUM_ITER):
    parts = []
    for c in range(NCHK):
      dst_c = lax.dynamic_slice_in_dim(dst, c * EC, EC)
      src_c = lax.dynamic_slice_in_dim(src, c * EC, EC)
      u, p = sc_gather(h, dst_c, src_c)
      msg = tc_gate(u, p, lnw2, lnb2, wg_bf, bg2)
      parts.append(sc_scatter(msg, dst_sca[c], z2))
    h = tc_update(*parts, degp_t, h, Wih, Whh, b2)
  return h


# speedup vs baseline: 1.0204x; 1.0204x over previous
import functools

import jax
import jax.numpy as jnp
from jax import lax
from jax.experimental import pallas as pl
from jax.experimental.pallas import tpu as pltpu
from jax.experimental.pallas import tpu_sc as plsc

N = 10000
E = 320000
D = 128
FD = 64
NUM_ITER = 2

NC = 2
NS = 16
L = 16
NW = NC * NS
NCHK = 5
EC = E // NCHK
EPW = EC // NW
CH = 40
SUB = 5
CB = CH * SUB
NSTEP = EPW // CB
NP2 = 10240
RPT = NP2 // NS
CH_S = 40
SUB_S = 5
CB_S = CH_S * SUB_S
NSTEP_S = EPW // CB_S
NP1 = 10240
DPT = NP1 // NS
EPW_D = E // NW
CH_D = 80
RPD = EPW_D // CH_D


def _sc_deg_body(dst2_ref, z1_ref, deg_out, idx_v, ones_v, deg_sp, sem):
  c = lax.axis_index("c")
  s = lax.axis_index("s")
  wid = s * NC + c
  pltpu.sync_copy(z1_ref.at[pl.ds(s * DPT, DPT)],
                  deg_sp.at[pl.ds(s * DPT, DPT)])
  for j in range(CH_D // L):
    ones_v[pl.ds(j * L, L)] = jnp.ones((L,), jnp.float32)
  pltpu.sync_copy(dst2_ref.at[wid], idx_v)
  plsc.subcore_barrier()

  def step(i, carry):
    copies = [pltpu.async_copy(ones_v, deg_sp.at[idx_v.at[i, k]],
                               sem, add=True)
              for k in range(SUB)]
    for cp in copies:
      cp.wait()
    return carry

  lax.fori_loop(0, RPD // SUB, step, 0)
  plsc.subcore_barrier()
  pltpu.sync_copy(deg_sp.at[pl.ds(s * DPT, DPT)],
                  deg_out.at[c, pl.ds(s * DPT, DPT)])


@functools.lru_cache(maxsize=None)
def _make_sc_deg():
  mesh = plsc.VectorSubcoreMesh(
      core_axis_name="c", subcore_axis_name="s", num_cores=NC,
      num_subcores=NS)
  return pl.kernel(
      _sc_deg_body,
      out_type=jax.ShapeDtypeStruct((NC, NP1), jnp.float32),
      mesh=mesh,
      scratch_types=[
          pltpu.VMEM((RPD // SUB, SUB, CH_D), jnp.int32),
          pltpu.VMEM((CH_D,), jnp.float32),
          pltpu.VMEM_SHARED((NP1,), jnp.float32),
          pltpu.SemaphoreType.DMA,
      ],
      compiler_params=pltpu.CompilerParams(use_tc_tiling_on_sc=True),
  )


def _sc_gather_body(tab_ref, dst_ref, src_ref, u_ref, p_ref,
                    idx_d0, idx_s0, idx_d1, idx_s1,
                    rows_d0, rows_s0, rows_d1, rows_s1,
                    sem_g0, sem_g1, sem_w0, sem_w1):
  wid = lax.axis_index("s") * NC + lax.axis_index("c")
  base0 = wid * EPW
  bufs = ((idx_d0, idx_s0, rows_d0, rows_s0, sem_g0, sem_w0),
          (idx_d1, idx_s1, rows_d1, rows_s1, sem_g1, sem_w1))

  def load_idx(i, b):
    idx_d, idx_s = bufs[b][0], bufs[b][1]
    base = base0 + i * CB
    pltpu.sync_copy(dst_ref.at[pl.ds(base, CB)], idx_d)
    pltpu.sync_copy(src_ref.at[pl.ds(base, CB)], idx_s)

  def fire_gathers(b):
    idx_d, idx_s, rows_d, rows_s, sem_g, _ = bufs[b]
    cps = []
    for k in range(SUB):
      cps.append(pltpu.async_copy(
          tab_ref.at[idx_d.at[pl.ds(k * CH, CH)]],
          rows_d.at[pl.ds(k * CH, CH)], sem_g))
      cps.append(pltpu.async_copy(
          tab_ref.at[idx_s.at[pl.ds(k * CH, CH)]],
          rows_s.at[pl.ds(k * CH, CH)], sem_g))
    return cps

  def fire_wb(i, b):
    _, _, rows_d, rows_s, _, sem_w = bufs[b]
    base = base0 + i * CB
    pltpu.async_copy(rows_d, u_ref.at[pl.ds(base, CB)], sem_w)
    pltpu.async_copy(rows_s, p_ref.at[pl.ds(base, CB)], sem_w)

  def drain_wb(b):
    _, _, rows_d, rows_s, _, sem_w = bufs[b]
    pltpu.make_async_copy(rows_d, u_ref.at[pl.ds(0, CB)], sem_w).wait()
    pltpu.make_async_copy(rows_s, p_ref.at[pl.ds(0, CB)], sem_w).wait()

  load_idx(0, 0)
  gcps = {0: fire_gathers(0)}
  for i in range(NSTEP):
    b = i % 2
    if i + 1 < NSTEP:
      if i >= 1:
        drain_wb(1 - b)
      load_idx(i + 1, 1 - b)
      gcps[i + 1] = fire_gathers(1 - b)
    for cp in gcps.pop(i):
      cp.wait()
    fire_wb(i, b)
  drain_wb(0)
  drain_wb(1)


@functools.lru_cache(maxsize=None)
def _make_sc_gather():
  mesh = plsc.VectorSubcoreMesh(
      core_axis_name="c", subcore_axis_name="s", num_cores=NC,
      num_subcores=NS)
  return pl.kernel(
      _sc_gather_body,
      out_type=(jax.ShapeDtypeStruct((EC, D), jnp.float32),
                jax.ShapeDtypeStruct((EC, D), jnp.float32)),
      mesh=mesh,
      scratch_types=[
          pltpu.VMEM((CB,), jnp.int32),
          pltpu.VMEM((CB,), jnp.int32),
          pltpu.VMEM((CB,), jnp.int32),
          pltpu.VMEM((CB,), jnp.int32),
          pltpu.VMEM((CB, D), jnp.float32),
          pltpu.VMEM((CB, D), jnp.float32),
          pltpu.VMEM((CB, D), jnp.float32),
          pltpu.VMEM((CB, D), jnp.float32),
          pltpu.SemaphoreType.DMA,
          pltpu.SemaphoreType.DMA,
          pltpu.SemaphoreType.DMA,
          pltpu.SemaphoreType.DMA,
      ],
      compiler_params=pltpu.CompilerParams(use_tc_tiling_on_sc=True),
  )


def _sc_scatter_body(msg_ref, dst2_ref, z2_ref, agg_out,
                     idx_v, msg_v, agg_sp, sem):
  c = lax.axis_index("c")
  s = lax.axis_index("s")
  wid = s * NC + c
  pltpu.sync_copy(z2_ref.at[pl.ds(s * RPT, RPT)],
                  agg_sp.at[pl.ds(s * RPT, RPT)])
  plsc.subcore_barrier()

  base0 = wid * EPW

  def step(i, carry):
    base = base0 + i * CB_S
    lds = [pltpu.async_copy(msg_ref.at[pl.ds(base, CB_S)], msg_v, sem),
           pltpu.async_copy(dst2_ref.at[wid, i], idx_v, sem)]
    for cp in lds:
      cp.wait()
    copies = [pltpu.async_copy(msg_v.at[pl.ds(k * CH_S, CH_S)],
                               agg_sp.at[idx_v.at[k]], sem, add=True)
              for k in range(SUB_S)]
    for cp in copies:
      cp.wait()
    return carry

  lax.fori_loop(0, NSTEP_S, step, 0)
  plsc.subcore_barrier()
  pltpu.sync_copy(agg_sp.at[pl.ds(s * RPT, RPT)],
                  agg_out.at[c, pl.ds(s * RPT, RPT)])


@functools.lru_cache(maxsize=None)
def _make_sc_scatter():
  mesh = plsc.VectorSubcoreMesh(
      core_axis_name="c", subcore_axis_name="s", num_cores=NC,
      num_subcores=NS)
  return pl.kernel(
      _sc_scatter_body,
      out_type=jax.ShapeDtypeStruct((NC, NP2, D), jnp.float32),
      mesh=mesh,
      scratch_types=[
          pltpu.VMEM((SUB_S, CH_S), jnp.int32),
          pltpu.VMEM((CB_S, D), jnp.float32),
          pltpu.VMEM_SHARED((NP2, D), jnp.float32),
          pltpu.SemaphoreType.DMA,
      ],
      compiler_params=pltpu.CompilerParams(use_tc_tiling_on_sc=True),
  )


BLK = 2560


def _tc_gate_body(u_ref, p_ref, lnw_ref, lnb_ref, wg_ref, bg_ref, msg_ref):
  u = u_ref[...]
  p = p_ref[...]
  s1 = jnp.sum(u, axis=1) + jnp.sum(p, axis=1)
  s2 = jnp.sum(u * u, axis=1) + jnp.sum(p * p, axis=1)
  mu = s1 / (2.0 * D)
  var = s2 / (2.0 * D) - mu * mu
  inv = lax.rsqrt(var + 1e-5)
  lnw = lnw_ref[...]
  lnb = lnb_ref[...]
  nu = jnp.maximum((u - mu[:, None]) * inv[:, None] * lnw[0] + lnb[0], 0.0)
  np_ = jnp.maximum((p - mu[:, None]) * inv[:, None] * lnw[1] + lnb[1], 0.0)
  wg = wg_ref[...]
  g = (jnp.dot(nu.astype(jnp.bfloat16), wg[:D],
               preferred_element_type=jnp.float32)
       + jnp.dot(np_.astype(jnp.bfloat16), wg[D:],
                 preferred_element_type=jnp.float32)
       + bg_ref[...])
  gate = jnp.mean(1.0 / (1.0 + jnp.exp(-g)), axis=1)
  msg_ref[...] = p * gate[:, None]


def _make_tc_gate():
  return pl.pallas_call(
      _tc_gate_body,
      grid=(EC // BLK,),
      in_specs=[
          pl.BlockSpec((BLK, D), lambda i: (i, 0)),
          pl.BlockSpec((BLK, D), lambda i: (i, 0)),
          pl.BlockSpec((2, D), lambda i: (0, 0)),
          pl.BlockSpec((2, D), lambda i: (0, 0)),
          pl.BlockSpec((2 * D, FD), lambda i: (0, 0)),
          pl.BlockSpec((1, FD), lambda i: (0, 0)),
      ],
      out_specs=pl.BlockSpec((BLK, D), lambda i: (i, 0)),
      out_shape=jax.ShapeDtypeStruct((EC, D), jnp.float32),
  )


RB = 2000


def _tc_update_body(p0, p1, p2, p3, p4, degp_ref, h_ref, wih_ref, whh_ref,
                    b_ref, hn_ref):
  agg = (p0[0] + p0[1] + p1[0] + p1[1] + p2[0] + p2[1]
         + p3[0] + p3[1] + p4[0] + p4[1])
  deg = degp_ref[:, 0] + degp_ref[:, 1]
  aggn = agg / jnp.maximum(deg, 1.0)[:, None]
  h = h_ref[...]
  hn = (jnp.dot(jnp.maximum(aggn, 0.0), wih_ref[...],
                preferred_element_type=jnp.float32)
        + jnp.dot(jnp.maximum(h, 0.0), whh_ref[...],
                  preferred_element_type=jnp.float32)
        + b_ref[0] + b_ref[1])
  hn_ref[...] = hn


def _make_tc_update():
  part_spec = pl.BlockSpec((NC, RB, D), lambda i: (0, i, 0))
  return pl.pallas_call(
      _tc_update_body,
      grid=(N // RB,),
      in_specs=[
          part_spec, part_spec, part_spec, part_spec, part_spec,
          pl.BlockSpec((RB, NC), lambda i: (i, 0)),
          pl.BlockSpec((RB, D), lambda i: (i, 0)),
          pl.BlockSpec((D, D), lambda i: (0, 0)),
          pl.BlockSpec((D, D), lambda i: (0, 0)),
          pl.BlockSpec((2, D), lambda i: (0, 0)),
      ],
      out_specs=pl.BlockSpec((RB, D), lambda i: (i, 0)),
      out_shape=jax.ShapeDtypeStruct((N, D), jnp.float32),
  )


def kernel(x, edge_index, ln_w, ln_b, Wg, bg, Wih, bih, Whh, bhh):
  src = edge_index[0]
  dst = edge_index[1]
  dst_deg = dst.reshape(NW, RPD // SUB, SUB, CH_D)
  dst_sca = dst.reshape(NCHK, NW, NSTEP_S, SUB_S, CH_S)
  lnw2 = ln_w.reshape(2, D)
  lnb2 = ln_b.reshape(2, D)
  wg_bf = Wg.astype(jnp.bfloat16)
  bg2 = bg.reshape(1, FD)
  b2 = jnp.stack([bih, bhh])
  z2 = jnp.zeros((NP2, D), jnp.float32)
  z1 = jnp.zeros((NP1,), jnp.float32)
  tc_gate = _make_tc_gate()
  tc_update = _make_tc_update()
  sc_gather = _make_sc_gather()
  sc_scatter = _make_sc_scatter()

  degp = _make_sc_deg()(dst_deg, z1)
  degp_t = degp.T[:N]

  h = x
  for _ in range(NUM_ITER):
    parts = []
    for c in range(NCHK):
      dst_c = lax.dynamic_slice_in_dim(dst, c * EC, EC)
      src_c = lax.dynamic_slice_in_dim(src, c * EC, EC)
      u, p = sc_gather(h, dst_c, src_c)
      msg = tc_gate(u, p, lnw2, lnb2, wg_bf, bg2)
      parts.append(sc_scatter(msg, dst_sca[c], z2))
    h = tc_update(*parts, degp_t, h, Wih, Whh, b2)
  return h
